# initial kernel scaffold (unmeasured)
import jax
import jax.numpy as jnp
from jax import lax
from jax.experimental import pallas as pl
from jax.experimental.pallas import tpu as pltpu

N_DEV = 8


def _gelu(y):
    c = 0.7978845608028654
    return 0.5 * y * (1.0 + jnp.tanh(c * (y + 0.044715 * y * y * y)))


def kernel(x, w_mat):
    m_per, k = x.shape
    _, n_per = w_mat.shape

    def body(x_ref, w_ref, out_ref, comm_ref, send_sems, recv_sems):
        my = lax.axis_index("i")
        left = lax.rem(my + N_DEV - 1, N_DEV)
        right = lax.rem(my + 1, N_DEV)

        barrier_sem = pltpu.get_barrier_semaphore()
        for nbr in (left, right):
            pl.semaphore_signal(
                barrier_sem, inc=1,
                device_id=(nbr,), device_id_type=pl.DeviceIdType.MESH,
            )
        pl.semaphore_wait(barrier_sem, 2)

        comm_ref[0] = x_ref[...]
        acc = jnp.dot(x_ref[...], w_ref[...], preferred_element_type=jnp.float32)
        out_ref[pl.ds(my * m_per, m_per), :] = _gelu(acc)

        for h in range(N_DEV - 1):
            send_slot = h % 2
            recv_slot = (h + 1) % 2
            rdma = pltpu.make_async_remote_copy(
                src_ref=comm_ref.at[send_slot],
                dst_ref=comm_ref.at[recv_slot],
                send_sem=send_sems.at[send_slot],
                recv_sem=recv_sems.at[recv_slot],
                device_id=(right,),
                device_id_type=pl.DeviceIdType.MESH,
            )
            rdma.start()
            rdma.wait()

            origin = lax.rem(my + N_DEV - 1 - h, N_DEV)
            acc = jnp.dot(
                comm_ref[recv_slot], w_ref[...],
                preferred_element_type=jnp.float32,
            )
            out_ref[pl.ds(origin * m_per, m_per), :] = _gelu(acc)

    return pl.pallas_call(
        body,
        out_shape=jax.ShapeDtypeStruct((N_DEV * m_per, n_per), jnp.float32),
        in_specs=[
            pl.BlockSpec(memory_space=pltpu.VMEM),
            pl.BlockSpec(memory_space=pltpu.VMEM),
        ],
        out_specs=pl.BlockSpec(memory_space=pltpu.VMEM),
        scratch_shapes=[
            pltpu.VMEM((2, m_per, k), x.dtype),
            pltpu.SemaphoreType.DMA((2,)),
            pltpu.SemaphoreType.DMA((2,)),
        ],
        compiler_params=pltpu.CompilerParams(collective_id=0),
    )(x, w_mat)


# baseline (device time: 1408829 ns/iter reference)
import jax
import jax.numpy as jnp
from jax import lax
from jax.experimental import pallas as pl
from jax.experimental.pallas import tpu as pltpu

N_DEV = 8


def _gelu(y):
    c = 0.7978845608028654
    return 0.5 * y * (1.0 + jnp.tanh(c * (y + 0.044715 * y * y * y)))


def kernel(x, w_mat):
    m_per, k = x.shape
    _, n_per = w_mat.shape

    x = x.astype(jnp.bfloat16)
    w_mat = w_mat.astype(jnp.bfloat16)

    def body(x_ref, w_ref, out_ref, comm_ref, send_sems, recv_sems, copy_sem):
        my = lax.axis_index("i")
        left = lax.rem(my + N_DEV - 1, N_DEV)
        right = lax.rem(my + 1, N_DEV)

        barrier_sem = pltpu.get_barrier_semaphore()
        for nbr in (left, right):
            pl.semaphore_signal(
                barrier_sem, inc=1,
                device_id=(nbr,), device_id_type=pl.DeviceIdType.MESH,
            )
        pl.semaphore_wait(barrier_sem, 2)

        own_copy = pltpu.make_async_copy(x_ref, comm_ref.at[0], copy_sem)
        own_copy.start()
        own_copy.wait()
        acc = jnp.dot(
            comm_ref[0], w_ref[...], preferred_element_type=jnp.float32
        )
        out_ref[pl.ds(my * m_per, m_per), :] = _gelu(acc)

        for h in range(N_DEV - 1):
            send_slot = h % 2
            recv_slot = (h + 1) % 2
            rdma = pltpu.make_async_remote_copy(
                src_ref=comm_ref.at[send_slot],
                dst_ref=comm_ref.at[recv_slot],
                send_sem=send_sems.at[send_slot],
                recv_sem=recv_sems.at[recv_slot],
                device_id=(right,),
                device_id_type=pl.DeviceIdType.MESH,
            )
            rdma.start()
            rdma.wait()

            origin = lax.rem(my + N_DEV - 1 - h, N_DEV)
            acc = jnp.dot(
                comm_ref[recv_slot], w_ref[...],
                preferred_element_type=jnp.float32,
            )
            out_ref[pl.ds(origin * m_per, m_per), :] = _gelu(acc)

    return pl.pallas_call(
        body,
        out_shape=jax.ShapeDtypeStruct((N_DEV * m_per, n_per), jnp.float32),
        in_specs=[
            pl.BlockSpec(memory_space=pltpu.MemorySpace.HBM),
            pl.BlockSpec(memory_space=pltpu.VMEM),
        ],
        out_specs=pl.BlockSpec(memory_space=pltpu.VMEM),
        scratch_shapes=[
            pltpu.VMEM((2, m_per, k), jnp.bfloat16),
            pltpu.SemaphoreType.DMA((2,)),
            pltpu.SemaphoreType.DMA((2,)),
            pltpu.SemaphoreType.DMA,
        ],
        compiler_params=pltpu.CompilerParams(
            collective_id=0, vmem_limit_bytes=63 * 1024 * 1024
        ),
    )(x, w_mat)


# device time: 709075 ns/iter; 1.9869x vs baseline; 1.9869x over previous
import jax
import jax.numpy as jnp
from jax import lax
from jax.experimental import pallas as pl
from jax.experimental.pallas import tpu as pltpu

N_DEV = 8
N_HOP = N_DEV - 1


def _gelu(y):
    c = 0.7978845608028654
    return 0.5 * y * (1.0 + jnp.tanh(c * (y + 0.044715 * y * y * y)))


def kernel(x, w_mat):
    m_per, k = x.shape
    _, n_per = w_mat.shape
    m_half = m_per // 2

    x = x.astype(jnp.bfloat16)
    w_mat = w_mat.astype(jnp.bfloat16)

    def body(x_ref, w_ref, out_ref, cw_ref, ccw_ref, stage_ref,
             cw_send, cw_recv, ccw_send, ccw_recv,
             credit_cw, credit_ccw, exit_sem, out_sems):
        my = lax.axis_index("i")
        left = lax.rem(my + N_DEV - 1, N_DEV)
        right = lax.rem(my + 1, N_DEV)

        barrier_sem = pltpu.get_barrier_semaphore()
        for nbr in (left, right):
            pl.semaphore_signal(
                barrier_sem, inc=1,
                device_id=(nbr,), device_id_type=pl.DeviceIdType.MESH,
            )
        pl.semaphore_wait(barrier_sem, 2)

        def make_send(direction, src, slot):
            comm, sends, recvs, dst_dev = (
                (cw_ref, cw_send, cw_recv, right) if direction == 0
                else (ccw_ref, ccw_send, ccw_recv, left)
            )
            return pltpu.make_async_remote_copy(
                src_ref=src,
                dst_ref=comm.at[slot],
                send_sem=sends.at[slot],
                recv_sem=recvs.at[slot],
                device_id=(dst_dev,),
                device_id_type=pl.DeviceIdType.MESH,
            )

        sends_cw = [make_send(0, x_ref.at[pl.ds(0, m_half)], 0)]
        sends_ccw = [make_send(1, x_ref.at[pl.ds(m_half, m_half)], 0)]
        sends_cw[0].start()
        sends_ccw[0].start()

        out_copies = [None, None]

        def emit(origin, half, stage_slot, acc):
            if out_copies[stage_slot] is not None:
                out_copies[stage_slot].wait()
            stage_ref[stage_slot] = _gelu(acc)
            row = origin * m_per + half * m_half
            cp = pltpu.make_async_copy(
                stage_ref.at[stage_slot],
                out_ref.at[pl.ds(row, m_half)],
                out_sems.at[stage_slot],
            )
            cp.start()
            out_copies[stage_slot] = cp

        emit(my, 0, 0, jnp.dot(x_ref[pl.ds(0, m_half)], w_ref[...],
                               preferred_element_type=jnp.float32))
        emit(my, 1, 1, jnp.dot(x_ref[pl.ds(m_half, m_half)], w_ref[...],
                               preferred_element_type=jnp.float32))

        for h in range(N_HOP):
            slot = h % 2
            sends_cw[h].wait_recv()
            sends_ccw[h].wait_recv()

            if h < N_HOP - 1:
                nslot = (h + 1) % 2
                if h >= 1:
                    pl.semaphore_wait(credit_cw, 1)
                    pl.semaphore_wait(credit_ccw, 1)
                sends_cw.append(make_send(0, cw_ref.at[slot], nslot))
                sends_ccw.append(make_send(1, ccw_ref.at[slot], nslot))
                sends_cw[h + 1].start()
                sends_ccw[h + 1].start()

            o_cw = lax.rem(my + N_DEV - 1 - h, N_DEV)
            o_ccw = lax.rem(my + 1 + h, N_DEV)
            emit(o_cw, 0, 0, jnp.dot(cw_ref[slot], w_ref[...],
                                     preferred_element_type=jnp.float32))
            emit(o_ccw, 1, 1, jnp.dot(ccw_ref[slot], w_ref[...],
                                      preferred_element_type=jnp.float32))

            if h < N_HOP - 1:
                if h == 0:
                    sends_cw[0].wait_send()
                    sends_ccw[0].wait_send()
                sends_cw[h + 1].wait_send()
                sends_ccw[h + 1].wait_send()
                if h <= N_HOP - 3:
                    pl.semaphore_signal(
                        credit_cw, inc=1,
                        device_id=(left,), device_id_type=pl.DeviceIdType.MESH,
                    )
                    pl.semaphore_signal(
                        credit_ccw, inc=1,
                        device_id=(right,), device_id_type=pl.DeviceIdType.MESH,
                    )

        out_copies[0].wait()
        out_copies[1].wait()

        for nbr in (left, right):
            pl.semaphore_signal(
                exit_sem, inc=1,
                device_id=(nbr,), device_id_type=pl.DeviceIdType.MESH,
            )
        pl.semaphore_wait(exit_sem, 2)

    return pl.pallas_call(
        body,
        out_shape=jax.ShapeDtypeStruct((N_DEV * m_per, n_per), jnp.float32),
        in_specs=[
            pl.BlockSpec(memory_space=pltpu.VMEM),
            pl.BlockSpec(memory_space=pltpu.VMEM),
        ],
        out_specs=pl.BlockSpec(memory_space=pltpu.MemorySpace.HBM),
        scratch_shapes=[
            pltpu.VMEM((2, m_half, k), jnp.bfloat16),
            pltpu.VMEM((2, m_half, k), jnp.bfloat16),
            pltpu.VMEM((2, m_half, n_per), jnp.float32),
            pltpu.SemaphoreType.DMA((2,)),
            pltpu.SemaphoreType.DMA((2,)),
            pltpu.SemaphoreType.DMA((2,)),
            pltpu.SemaphoreType.DMA((2,)),
            pltpu.SemaphoreType.REGULAR,
            pltpu.SemaphoreType.REGULAR,
            pltpu.SemaphoreType.REGULAR,
            pltpu.SemaphoreType.DMA((2,)),
        ],
        compiler_params=pltpu.CompilerParams(
            collective_id=0, vmem_limit_bytes=63 * 1024 * 1024
        ),
    )(x, w_mat)
